# baseline (speedup 1.0000x reference)
import functools

import jax
import jax.numpy as jnp
from jax import lax
from jax.experimental import pallas as pl
from jax.experimental.pallas import tpu as pltpu
from jax.experimental.pallas import tpu_sc as plsc

D = 1024
L = 16
NC, NS = 2, 16
NW = NC * NS
C = 16


def _rsqrt_vec(x):
    i = lax.bitcast_convert_type(x, jnp.int32)
    i = jnp.int32(0x5F3759DF) - (i >> 1)
    y = lax.bitcast_convert_type(i, jnp.float32)
    for _ in range(3):
        y = y * (1.5 - 0.5 * x * y * y)
    return y


def _make_sc_kernel(B):
    rows_per_w = B // NW
    nchunk = rows_per_w // C
    npair = nchunk // 2
    mesh = plsc.VectorSubcoreMesh(core_axis_name="c", subcore_axis_name="s")

    @functools.partial(
        pl.kernel,
        mesh=mesh,
        out_type=jax.ShapeDtypeStruct((B, D), jnp.float32),
        scratch_types=[
            pltpu.VMEM((rows_per_w,), jnp.int32),
            pltpu.VMEM((C, D), jnp.float32),
            pltpu.VMEM((C, D), jnp.float32),
            pltpu.VMEM((C, D), jnp.float32),
            pltpu.VMEM((C, D), jnp.float32),
            pltpu.SemaphoreType.DMA,
            pltpu.SemaphoreType.DMA,
            pltpu.SemaphoreType.DMA,
            pltpu.SemaphoreType.DMA,
        ],
    )
    def k(pos_hbm, table_hbm, out_hbm, idx_v, bin0, bin1, bout0, bout1,
          gs0, gs1, ss0, ss1):
        wid = lax.axis_index("s") * NC + lax.axis_index("c")
        row0 = wid * rows_per_w
        pltpu.sync_copy(pos_hbm.at[pl.ds(row0, rows_per_w)], idx_v)

        bins, bouts = (bin0, bin1), (bout0, bout1)
        gsems, ssems = (gs0, gs1), (ss0, ss1)

        def gather_start(c, b):
            pltpu.async_copy(
                table_hbm.at[idx_v.at[pl.ds(c * C, C)]], bins[b], gsems[b])

        def gather_wait(b):
            pltpu.make_async_copy(
                table_hbm.at[idx_v.at[pl.ds(0, C)]], bins[b], gsems[b]).wait()

        def scatter_start(c, b):
            pltpu.async_copy(
                bouts[b], out_hbm.at[pl.ds(row0 + c * C, C)], ssems[b])

        def scatter_wait(b):
            pltpu.make_async_copy(
                bouts[b], out_hbm.at[pl.ds(row0, C)], ssems[b]).wait()

        def compute(b):
            src, dst = bins[b], bouts[b]
            lane = lax.iota(jnp.int32, L)

            def sumsq(r):
                accs = [jnp.zeros((L,), jnp.float32) for _ in range(8)]
                for j in range(D // L):
                    v = src[r, pl.ds(j * L, L)]
                    accs[j % 8] = accs[j % 8] + v * v
                acc01 = accs[0] + accs[1]
                acc23 = accs[2] + accs[3]
                acc45 = accs[4] + accs[5]
                acc67 = accs[6] + accs[7]
                acc = (acc01 + acc23) + (acc45 + acc67)
                for kk in (8, 4, 2, 1):
                    perm = jnp.bitwise_xor(lane, kk)
                    acc = acc + acc.at[perm].get(mode="promise_in_bounds")
                return acc

            def row_body(rr, carry):
                r0 = rr * 2
                r1 = r0 + 1
                inv0 = _rsqrt_vec(jnp.maximum(sumsq(r0), 1e-24))
                inv1 = _rsqrt_vec(jnp.maximum(sumsq(r1), 1e-24))
                for j in range(D // L):
                    dst[r0, pl.ds(j * L, L)] = src[r0, pl.ds(j * L, L)] * inv0
                    dst[r1, pl.ds(j * L, L)] = src[r1, pl.ds(j * L, L)] * inv1
                return carry

            lax.fori_loop(0, C // 2, row_body, 0)

        gather_start(0, 0)
        gather_start(1, 1)

        def pair_body(p, carry):
            for b in (0, 1):
                c = 2 * p + b
                gather_wait(b)

                @pl.when(p > 0)
                def _():
                    scatter_wait(b)

                compute(b)
                scatter_start(c, b)

                @pl.when(p < npair - 1)
                def _():
                    gather_start(c + 2, b)

            return carry

        lax.fori_loop(0, npair, pair_body, 0)
        scatter_wait(0)
        scatter_wait(1)

    return k


def kernel(positions, positional_embeddings):
    B = positions.size
    pos_flat = positions.reshape(-1).astype(jnp.int32)
    table = positional_embeddings.astype(jnp.float32)
    out = _make_sc_kernel(B)(pos_flat, table)
    return out.reshape(positions.shape + (D,))


# speedup vs baseline: 1.1855x; 1.1855x over previous
import functools

import jax
import jax.numpy as jnp
from jax import lax
from jax.experimental import pallas as pl
from jax.experimental.pallas import tpu as pltpu
from jax.experimental.pallas import tpu_sc as plsc

D = 1024
L = 16
NC, NS = 2, 16
NW = NC * NS
C = 16


def _rsqrt_vec(x):
    i = lax.bitcast_convert_type(x, jnp.int32)
    i = jnp.int32(0x5F3759DF) - (i >> 1)
    y = lax.bitcast_convert_type(i, jnp.float32)
    for _ in range(3):
        y = y * (1.5 - 0.5 * x * y * y)
    return y


def _make_sc_kernel(B):
    rows_per_w = B // NW
    nchunk = rows_per_w // C
    npair = nchunk // 2
    mesh = plsc.VectorSubcoreMesh(core_axis_name="c", subcore_axis_name="s")

    @functools.partial(
        pl.kernel,
        mesh=mesh,
        out_type=jax.ShapeDtypeStruct((B, D), jnp.float32),
        scratch_types=[
            pltpu.VMEM((rows_per_w,), jnp.int32),
            pltpu.VMEM((C, D), jnp.float32),
            pltpu.VMEM((C, D), jnp.float32),
            pltpu.VMEM((C, D), jnp.float32),
            pltpu.VMEM((C, D), jnp.float32),
            pltpu.VMEM((C, L), jnp.float32),
            pltpu.SemaphoreType.DMA,
            pltpu.SemaphoreType.DMA,
            pltpu.SemaphoreType.DMA,
            pltpu.SemaphoreType.DMA,
        ],
    )
    def k(pos_hbm, table_hbm, out_hbm, idx_v, bin0, bin1, bout0, bout1,
          accbuf, gs0, gs1, ss0, ss1):
        wid = lax.axis_index("s") * NC + lax.axis_index("c")
        row0 = wid * rows_per_w
        pltpu.sync_copy(pos_hbm.at[pl.ds(row0, rows_per_w)], idx_v)

        bins, bouts = (bin0, bin1), (bout0, bout1)
        gsems, ssems = (gs0, gs1), (ss0, ss1)

        def gather_start(c, b):
            pltpu.async_copy(
                table_hbm.at[idx_v.at[pl.ds(c * C, C)]], bins[b], gsems[b])

        def gather_wait(b):
            pltpu.make_async_copy(
                table_hbm.at[idx_v.at[pl.ds(0, C)]], bins[b], gsems[b]).wait()

        def scatter_start(c, b):
            pltpu.async_copy(
                bouts[b], out_hbm.at[pl.ds(row0 + c * C, C)], ssems[b])

        def scatter_wait(b):
            pltpu.make_async_copy(
                bouts[b], out_hbm.at[pl.ds(row0, C)], ssems[b]).wait()

        lane = lax.iota(jnp.int32, L)

        def compute(b):
            src, dst = bins[b], bouts[b]

            def sumsq_body(r, carry):
                accs = [jnp.zeros((L,), jnp.float32) for _ in range(8)]
                for j in range(D // L):
                    v = src[r, pl.ds(j * L, L)]
                    accs[j % 8] = accs[j % 8] + v * v
                acc01 = accs[0] + accs[1]
                acc23 = accs[2] + accs[3]
                acc45 = accs[4] + accs[5]
                acc67 = accs[6] + accs[7]
                accbuf[r, :] = (acc01 + acc23) + (acc45 + acc67)
                return carry

            lax.fori_loop(0, C, sumsq_body, 0)

            vecs = [accbuf[r, :] for r in range(C)]
            for o in (1, 2, 4, 8):
                perm = jnp.bitwise_xor(lane, o)
                pick_b = (jnp.bitwise_and(lane, o) != 0)
                nxt = []
                for i in range(0, len(vecs), 2):
                    a, bb = vecs[i], vecs[i + 1]
                    asum = a + a.at[perm].get(mode="promise_in_bounds")
                    bsum = bb + bb.at[perm].get(mode="promise_in_bounds")
                    nxt.append(jnp.where(pick_b, bsum, asum))
                vecs = nxt
            inv_vec = _rsqrt_vec(jnp.maximum(vecs[0], 1e-24))

            def scale_body(r, inv_v):
                idxr = jnp.full((L,), r, jnp.int32)
                inv = inv_v.at[idxr].get(mode="promise_in_bounds")
                for j in range(D // L):
                    dst[r, pl.ds(j * L, L)] = src[r, pl.ds(j * L, L)] * inv
                return inv_v

            lax.fori_loop(0, C, scale_body, inv_vec)

        gather_start(0, 0)
        gather_start(1, 1)

        def pair_body(p, carry):
            for b in (0, 1):
                c = 2 * p + b
                gather_wait(b)

                @pl.when(p > 0)
                def _():
                    scatter_wait(b)

                compute(b)
                scatter_start(c, b)

                @pl.when(p < npair - 1)
                def _():
                    gather_start(c + 2, b)

            return carry

        lax.fori_loop(0, npair, pair_body, 0)
        scatter_wait(0)
        scatter_wait(1)

    return k


def kernel(positions, positional_embeddings):
    B = positions.size
    pos_flat = positions.reshape(-1).astype(jnp.int32)
    table = positional_embeddings.astype(jnp.float32)
    out = _make_sc_kernel(B)(pos_flat, table)
    return out.reshape(positions.shape + (D,))
